# baseline (speedup 1.0000x reference)
import functools

import jax
import jax.numpy as jnp
from jax import lax
from jax.experimental import pallas as pl
from jax.experimental.pallas import tpu as pltpu
from jax.experimental.pallas import tpu_sc as plsc

N_ROWS = 18432
N_STATES = 1024
Z_DIM = 64
BLOCK = 1024
N_CHUNKS = 2
CHUNK_ROWS = N_ROWS // N_CHUNKS
CHUNK_BLOCKS = CHUNK_ROWS // BLOCK
COMMITMENT_COST = 0.25

_SC_INFO = plsc.get_sparse_core_info()
N_WORKERS = _SC_INFO.num_cores * _SC_INFO.num_subcores
RPW = CHUNK_ROWS // N_WORKERS
IDX_TILE = 96
N_IDX_TILES = RPW // IDX_TILE


def _tc_kernel(chunk, x_ref, w_ref, cin_ref,
               enc_ref, idx_ref, cout_ref, perp_ref,
               wn_ref, iota_ref, counts_ref):
    del chunk
    i = pl.program_id(0)
    x = x_ref[...]
    w = w_ref[...]

    @pl.when(i == 0)
    def _init():
        wn_ref[...] = jnp.sum(w * w, axis=1).reshape(1, N_STATES)
        iota_ref[...] = jax.lax.broadcasted_iota(
            jnp.int32, (1, N_STATES), 1).astype(jnp.float32)
        counts_ref[...] = cin_ref[...]

    rn = jnp.sum(x * x, axis=1, keepdims=True)
    mm2 = jax.lax.dot_general(x + x, w, (((1,), (1,)), ((), ())),
                              preferred_element_type=jnp.float32)
    d = rn + wn_ref[...] - mm2

    m = jnp.min(d, axis=1, keepdims=True)
    ii = iota_ref[...]
    idxf = jnp.min(jnp.where(d == m, ii, jnp.float32(N_STATES)),
                   axis=1, keepdims=True)
    onehot = (ii == idxf).astype(jnp.float32)
    enc_ref[...] = onehot
    idx_ref[...] = idxf

    ones_row = jnp.ones((1, BLOCK), jnp.float32)
    counts_ref[...] += jax.lax.dot_general(
        ones_row, onehot, (((1,), (0,)), ((), ())),
        preferred_element_type=jnp.float32)

    @pl.when(i == CHUNK_BLOCKS - 1)
    def _fini():
        cout_ref[...] = counts_ref[...]
        avg = counts_ref[...] / N_ROWS
        ent = jnp.sum(avg * jnp.log(avg + 1e-10), keepdims=True)
        perp_ref[...] = jnp.exp(-ent)


def _tc_call(chunk, x, w, counts_in, enc_prev):
    base = chunk * CHUNK_BLOCKS
    in_specs = [
        pl.BlockSpec((BLOCK, Z_DIM), lambda i: (base + i, 0)),
        pl.BlockSpec((N_STATES, Z_DIM), lambda i: (0, 0)),
        pl.BlockSpec((1, N_STATES), lambda i: (0, 0)),
    ]
    args = [x, w, counts_in]
    io_aliases = {}
    if enc_prev is not None:
        in_specs.append(pl.BlockSpec(memory_space=pl.ANY))
        args.append(enc_prev)
        io_aliases = {3: 0}

    def body(*refs):
        if enc_prev is not None:
            refs = refs[:3] + refs[4:]
        return _tc_kernel(chunk, *refs)

    return pl.pallas_call(
        body,
        grid=(CHUNK_BLOCKS,),
        in_specs=in_specs,
        out_specs=[
            pl.BlockSpec((BLOCK, N_STATES), lambda i: (base + i, 0)),
            pl.BlockSpec((BLOCK, 1), lambda i: (i, 0)),
            pl.BlockSpec((1, N_STATES), lambda i: (0, 0)),
            pl.BlockSpec((1, 1), lambda i: (0, 0)),
        ],
        out_shape=[
            jax.ShapeDtypeStruct((N_ROWS, N_STATES), jnp.float32),
            jax.ShapeDtypeStruct((CHUNK_ROWS, 1), jnp.float32),
            jax.ShapeDtypeStruct((1, N_STATES), jnp.float32),
            jax.ShapeDtypeStruct((1, 1), jnp.float32),
        ],
        scratch_shapes=[
            pltpu.VMEM((1, N_STATES), jnp.float32),
            pltpu.VMEM((1, N_STATES), jnp.float32),
            pltpu.VMEM((1, N_STATES), jnp.float32),
        ],
        input_output_aliases=io_aliases,
    )(*args)


def _sc_body(chunk, w_hbm, idx_hbm, x_hbm, qst_hbm, sse_hbm,
             idx_v, rows_v, x_v, qst_v, acc_v, sem):
    nc = _SC_INFO.num_cores
    wid = lax.axis_index("s") * nc + lax.axis_index("c")
    base = wid * RPW
    gbase = chunk * CHUNK_ROWS + base

    for k in range(N_IDX_TILES):
        pltpu.sync_copy(idx_hbm.at[pl.ds(base + k * IDX_TILE, IDX_TILE)],
                        idx_v.at[k])
    copies = [
        pltpu.async_copy(w_hbm.at[idx_v.at[k]],
                         rows_v.at[pl.ds(k * IDX_TILE, IDX_TILE)], sem)
        for k in range(N_IDX_TILES)
    ]
    pltpu.sync_copy(x_hbm.at[pl.ds(gbase, RPW)], x_v)
    for c in copies:
        c.wait()

    def row(r, acc):
        accs = []
        for j in range(Z_DIM // 16):
            xv = x_v[r, pl.ds(j * 16, 16)]
            qv = rows_v[r, pl.ds(j * 16, 16)]
            dq = qv - xv
            qst_v[r, pl.ds(j * 16, 16)] = xv + dq
            accs.append(dq * dq)
        return acc + accs[0] + accs[1] + accs[2] + accs[3]

    acc = lax.fori_loop(0, RPW, row, jnp.zeros((16,), jnp.float32))
    acc_v[...] = acc
    pltpu.sync_copy(qst_v, qst_hbm.at[pl.ds(base, RPW)])
    pltpu.sync_copy(acc_v, sse_hbm.at[wid])


def _sc_call(chunk, w, idx_i32, x):
    mesh = plsc.VectorSubcoreMesh(core_axis_name="c", subcore_axis_name="s")
    fn = pl.kernel(
        functools.partial(_sc_body, chunk),
        mesh=mesh,
        out_type=[
            jax.ShapeDtypeStruct((CHUNK_ROWS, Z_DIM), jnp.float32),
            jax.ShapeDtypeStruct((N_WORKERS, 16), jnp.float32),
        ],
        scratch_types=[
            pltpu.VMEM((N_IDX_TILES, IDX_TILE), jnp.int32),
            pltpu.VMEM((RPW, 128), jnp.float32),
            pltpu.VMEM((RPW, Z_DIM), jnp.float32),
            pltpu.VMEM((RPW, Z_DIM), jnp.float32),
            pltpu.VMEM((16,), jnp.float32),
            pltpu.SemaphoreType.DMA,
        ],
    )
    return fn(w, idx_i32, x)


@jax.jit
def kernel(inputs, weight):
    counts0 = jnp.zeros((1, N_STATES), jnp.float32)
    enc0, idxf0, counts1, _ = _tc_call(0, inputs, weight, counts0, None)
    idx0 = idxf0.reshape(CHUNK_ROWS).astype(jnp.int32)
    enc, idxf1, _, perp = _tc_call(1, inputs, weight, counts1, enc0)
    idx1 = idxf1.reshape(CHUNK_ROWS).astype(jnp.int32)

    w_pad = jnp.concatenate([weight, jnp.zeros_like(weight)], axis=1)
    qst0, sse_p0 = _sc_call(0, w_pad, idx0, inputs)
    qst1, sse_p1 = _sc_call(1, w_pad, idx1, inputs)
    qst = jnp.concatenate([qst0, qst1], axis=0)

    sse = jnp.sum(sse_p0) + jnp.sum(sse_p1)
    loss = (1.0 + COMMITMENT_COST) * sse / (N_ROWS * Z_DIM)
    return (loss, qst, perp.reshape(()), enc)


# speedup vs baseline: 1.6665x; 1.6665x over previous
import jax
import jax.numpy as jnp
from jax.experimental import pallas as pl
from jax.experimental.pallas import tpu as pltpu

N_ROWS = 18432
N_STATES = 1024
Z_DIM = 64
BLOCK = 1024
N_BLOCKS = N_ROWS // BLOCK
N_GRID = N_BLOCKS + 1
COMMITMENT_COST = 0.25


def _vq_kernel(x_mm_ref, x_q_ref, w_ref,
               loss_ref, q_ref, perp_ref, enc_ref,
               mm2_ref, rn_ref, wn_ref, iota_ref, counts_ref, sse_ref):
    s = pl.program_id(0)
    w = w_ref[...]

    @pl.when(s == 0)
    def _init():
        wn_ref[...] = jnp.sum(w * w, axis=1).reshape(1, N_STATES)
        iota_ref[...] = jax.lax.broadcasted_iota(
            jnp.int32, (1, N_STATES), 1).astype(jnp.float32)
        counts_ref[...] = jnp.zeros_like(counts_ref)
        sse_ref[...] = jnp.zeros_like(sse_ref)
        mm2_ref[...] = jnp.zeros_like(mm2_ref)
        rn_ref[...] = jnp.zeros_like(rn_ref)

    d = rn_ref[...] + wn_ref[...] - mm2_ref[...]
    m = jnp.min(d, axis=1, keepdims=True)
    ii = iota_ref[...]
    idx = jnp.min(jnp.where(d == m, ii, jnp.float32(N_STATES)),
                  axis=1, keepdims=True)
    onehot = (ii == idx).astype(jnp.float32)
    enc_ref[...] = onehot

    xq = x_q_ref[...]
    q = jax.lax.dot_general(onehot, w, (((1,), (0,)), ((), ())),
                            preferred_element_type=jnp.float32)
    dq = q - xq
    q_ref[...] = xq + dq

    live = s >= 1
    ones_row = jnp.ones((1, BLOCK), jnp.float32)
    counts_ref[...] += jnp.where(
        live,
        jax.lax.dot_general(ones_row, onehot, (((1,), (0,)), ((), ())),
                            preferred_element_type=jnp.float32),
        0.0)
    sse_ref[...] += jnp.where(live, jnp.sum(dq * dq, keepdims=True), 0.0)

    x1 = x_mm_ref[...]
    rn_ref[...] = jnp.sum(x1 * x1, axis=1, keepdims=True)
    mm2_ref[...] = jax.lax.dot_general(
        x1 + x1, w, (((1,), (1,)), ((), ())),
        preferred_element_type=jnp.float32)

    @pl.when(s == N_GRID - 1)
    def _fini():
        sse = sse_ref[0, 0]
        loss_ref[...] = jnp.full((1, 1), (1.0 + COMMITMENT_COST)
                                 * sse / (N_ROWS * Z_DIM))
        avg = counts_ref[...] / N_ROWS
        ent = jnp.sum(avg * jnp.log(avg + 1e-10), keepdims=True)
        perp_ref[...] = jnp.exp(-ent)


@jax.jit
def kernel(inputs, weight):
    last = N_BLOCKS - 1
    loss, quantized_st, perp, encodings = pl.pallas_call(
        _vq_kernel,
        grid=(N_GRID,),
        in_specs=[
            pl.BlockSpec((BLOCK, Z_DIM),
                         lambda s: (jnp.minimum(s, last), 0)),
            pl.BlockSpec((BLOCK, Z_DIM),
                         lambda s: (jnp.clip(s - 1, 0, last), 0)),
            pl.BlockSpec((N_STATES, Z_DIM), lambda s: (0, 0)),
        ],
        out_specs=[
            pl.BlockSpec((1, 1), lambda s: (0, 0)),
            pl.BlockSpec((BLOCK, Z_DIM),
                         lambda s: (jnp.clip(s - 1, 0, last), 0)),
            pl.BlockSpec((1, 1), lambda s: (0, 0)),
            pl.BlockSpec((BLOCK, N_STATES),
                         lambda s: (jnp.clip(s - 1, 0, last), 0)),
        ],
        out_shape=[
            jax.ShapeDtypeStruct((1, 1), jnp.float32),
            jax.ShapeDtypeStruct((N_ROWS, Z_DIM), jnp.float32),
            jax.ShapeDtypeStruct((1, 1), jnp.float32),
            jax.ShapeDtypeStruct((N_ROWS, N_STATES), jnp.float32),
        ],
        scratch_shapes=[
            pltpu.VMEM((BLOCK, N_STATES), jnp.float32),
            pltpu.VMEM((BLOCK, 1), jnp.float32),
            pltpu.VMEM((1, N_STATES), jnp.float32),
            pltpu.VMEM((1, N_STATES), jnp.float32),
            pltpu.VMEM((1, N_STATES), jnp.float32),
            pltpu.VMEM((1, 1), jnp.float32),
        ],
    )(inputs, inputs, weight)
    return (loss.reshape(()), quantized_st, perp.reshape(()), encodings)
